# baseline (speedup 1.0000x reference)
import jax
import jax.numpy as jnp
from jax import lax
from jax.experimental import pallas as pl
from jax.experimental.pallas import tpu as pltpu
from jax.experimental.pallas import tpu_sc as plsc

_NCLS = 256
_NF = 256
_NPIX = 16384
_PPT = 1024
_P1 = 128
_G1 = _P1 // 16
_PC = 2048
_FPT = 16
_FH = 8


def _sc_body(lg_hbm, emb_hbm,
             sem_hbm, sum_hbm, sq_hbm, cnt_hbm,
             lg_v, emb_v, semt_v, semc_v,
             bins_s, bins_q, cnt_bins,
             comp_s, comp_q, cnt_c):
    c = lax.axis_index("c")
    s = lax.axis_index("s")
    lane = lax.iota(jnp.int32, 16)
    zf = jnp.zeros((16,), jnp.float32)
    onef = jnp.ones((16,), jnp.float32)

    def _z0(i, _):
        cnt_bins[pl.ds(i * 16, 16)] = zf
        return 0
    lax.fori_loop(0, _NCLS * 16 // 16, _z0, 0)

    base_pix = s * _PPT

    def chunk_body(k, _):
        p0 = base_pix + k * _P1
        pltpu.sync_copy(lg_hbm.at[c, :, pl.ds(p0, _P1)], lg_v)

        bv = tuple(lg_v[0, pl.ds(g * 16, 16)] for g in range(_G1))
        bi = tuple(jnp.zeros((16,), jnp.int32) for _ in range(_G1))

        def cls_body(cls, carry):
            pbv, pbi = carry
            civ = jnp.full((16,), cls, jnp.int32)
            nbv, nbi = [], []
            for g in range(_G1):
                v = lg_v[cls, pl.ds(g * 16, 16)]
                m = v > pbv[g]
                nbv.append(jnp.where(m, v, pbv[g]))
                nbi.append(jnp.where(m, civ, pbi[g]))
            return tuple(nbv), tuple(nbi)

        bv, bi = lax.fori_loop(1, _NCLS, cls_body, (bv, bi))
        for g in range(_G1):
            semt_v[pl.ds(k * _P1 + g * 16, 16)] = bi[g]
            plsc.addupdate_scatter(cnt_bins, [bi[g] * 16 + lane], onef)
        return 0

    lax.fori_loop(0, _PPT // _P1, chunk_body, 0)

    pltpu.sync_copy(semt_v, sem_hbm.at[pl.ds(c * _NPIX + s * _PPT, _PPT)])

    for cb in range(_NCLS // 16):
        cidx = (jnp.full((16,), cb * 16, jnp.int32) + lane) * 16
        acc = zf
        for l in range(16):
            acc = acc + plsc.load_gather(cnt_bins, [cidx + l])
        cnt_c[pl.ds(cb * 16, 16)] = acc
    pltpu.sync_copy(cnt_c, cnt_hbm.at[pl.ds((c * 16 + s) * _NCLS, _NCLS)])

    plsc.subcore_barrier()

    pat = (lane // 8) * (_FH * 16) + (lane % 8) * 16

    for ph in range(2):
        fbase = s * _FPT + ph * _FH

        def _z1(i, _):
            bins_s[pl.ds(i * 16, 16)] = zf
            bins_q[pl.ds(i * 16, 16)] = zf
            return 0
        lax.fori_loop(0, _NCLS * _FH, _z1, 0)

        def pix_body(pc, _):
            p0 = pc * _PC
            pltpu.sync_copy(sem_hbm.at[pl.ds(c * _NPIX + p0, _PC)], semc_v)
            pltpu.sync_copy(emb_hbm.at[c, pl.ds(fbase, _FH), pl.ds(p0, _PC)],
                            emb_v)

            def g_body(g, _):
                idx = semc_v[pl.ds(g * 16, 16)]
                base = idx * (_FH * 16) + lane
                for f in range(_FH):
                    v = emb_v[f, pl.ds(g * 16, 16)]
                    fidx = base + f * 16
                    plsc.addupdate_scatter(bins_s, [fidx], v)
                    plsc.addupdate_scatter(bins_q, [fidx], v * v)
                return 0

            lax.fori_loop(0, _PC // 16, g_body, 0)
            return 0

        lax.fori_loop(0, _NPIX // _PC, pix_body, 0)

        def red_body(cb, _):
            bidx = cb * (2 * _FH * 16) + pat
            acc_s = zf
            acc_q = zf
            for l in range(16):
                acc_s = acc_s + plsc.load_gather(bins_s, [bidx + l])
                acc_q = acc_q + plsc.load_gather(bins_q, [bidx + l])
            fi = ph * _FH + lane % 8
            ci = cb * 2 + lane // 8
            plsc.store_scatter(comp_s, [fi, ci], acc_s)
            plsc.store_scatter(comp_q, [fi, ci], acc_q)
            return 0

        lax.fori_loop(0, _NCLS // 2, red_body, 0)

    pltpu.sync_copy(comp_s, sum_hbm.at[c, pl.ds(s * _FPT, _FPT), :])
    pltpu.sync_copy(comp_q, sq_hbm.at[c, pl.ds(s * _FPT, _FPT), :])


def _epi_body(it_ref, s_ref, q_ref, cnt_ref, feat_ref, ex_ref, ex2_ref,
              cntin_ref, nf_ref, nex_ref, nex2_ref, ncnt_ref):
    train = (it_ref[0] != 0).astype(jnp.float32)
    S = jnp.transpose(s_ref[0] + s_ref[1])
    Q = jnp.transpose(q_ref[0] + q_ref[1])
    cnew_row = jnp.sum(cnt_ref[...], axis=0, keepdims=True)
    cold_row = cntin_ref[...]
    ones_row = jnp.ones((1, _NF), jnp.float32)
    dn = (((0,), (0,)), ((), ()))
    cnew = lax.dot_general(cnew_row, ones_row, dn,
                           preferred_element_type=jnp.float32)
    cold = lax.dot_general(cold_row, ones_row, dn,
                           preferred_element_type=jnp.float32)
    feat = feat_ref[...]
    upd_f = (feat * cold + S) / (cold + cnew + 1e-8)
    nf_ref[...] = train * upd_f + (1.0 - train) * feat
    nex_ref[...] = ex_ref[...] + train * S
    nex2_ref[...] = ex2_ref[...] + train * Q
    ncnt_ref[...] = cold_row + train * cnew_row


def kernel(unified_embedding, logits, gt, is_train, dataset_ids, features, ex,
           ex2, count):
    B, C, H, W = unified_embedding.shape
    L = logits.shape[1]
    N = H * W
    emb = unified_embedding.reshape(B, C, N)
    lg = logits.reshape(B, L, N)

    mesh = plsc.VectorSubcoreMesh(core_axis_name="c", subcore_axis_name="s")
    sc = pl.kernel(
        _sc_body,
        out_type=[
            jax.ShapeDtypeStruct((B * N,), jnp.int32),
            jax.ShapeDtypeStruct((B, C, L), jnp.float32),
            jax.ShapeDtypeStruct((B, C, L), jnp.float32),
            jax.ShapeDtypeStruct((B * 16 * L,), jnp.float32),
        ],
        mesh=mesh,
        compiler_params=pltpu.CompilerParams(needs_layout_passes=False),
        scratch_types=[
            pltpu.VMEM((_NCLS, _P1), jnp.float32),
            pltpu.VMEM((_FH, _PC), jnp.float32),
            pltpu.VMEM((_PPT,), jnp.int32),
            pltpu.VMEM((_PC,), jnp.int32),
            pltpu.VMEM((_NCLS * _FH * 16,), jnp.float32),
            pltpu.VMEM((_NCLS * _FH * 16,), jnp.float32),
            pltpu.VMEM((_NCLS * 16,), jnp.float32),
            pltpu.VMEM((_FPT, _NCLS), jnp.float32),
            pltpu.VMEM((_FPT, _NCLS), jnp.float32),
            pltpu.VMEM((_NCLS,), jnp.float32),
        ],
    )
    _sem, sums, sqs, cnts = sc(lg, emb)

    it = jnp.asarray(is_train, jnp.int32).reshape(1)
    out = pl.pallas_call(
        _epi_body,
        in_specs=[
            pl.BlockSpec(memory_space=pltpu.SMEM),
            pl.BlockSpec((B, C, L), lambda: (0, 0, 0)),
            pl.BlockSpec((B, C, L), lambda: (0, 0, 0)),
            pl.BlockSpec((B * 16, L), lambda: (0, 0)),
            pl.BlockSpec((L, C), lambda: (0, 0)),
            pl.BlockSpec((L, C), lambda: (0, 0)),
            pl.BlockSpec((L, C), lambda: (0, 0)),
            pl.BlockSpec((1, L), lambda: (0, 0)),
        ],
        out_specs=[
            pl.BlockSpec((L, C), lambda: (0, 0)),
            pl.BlockSpec((L, C), lambda: (0, 0)),
            pl.BlockSpec((L, C), lambda: (0, 0)),
            pl.BlockSpec((1, L), lambda: (0, 0)),
        ],
        out_shape=[
            jax.ShapeDtypeStruct((L, C), jnp.float32),
            jax.ShapeDtypeStruct((L, C), jnp.float32),
            jax.ShapeDtypeStruct((L, C), jnp.float32),
            jax.ShapeDtypeStruct((1, L), jnp.float32),
        ],
    )(it, sums, sqs, cnts.reshape(B * 16, L), features, ex, ex2,
      count.reshape(1, L))

    new_features, new_ex, new_ex2, new_count = out
    acc_loss = jnp.zeros((), jnp.float32)
    return (acc_loss, new_features, new_ex, new_ex2, new_count.reshape(L))


# speedup vs baseline: 1.0417x; 1.0417x over previous
import jax
import jax.numpy as jnp
from jax import lax
from jax.experimental import pallas as pl
from jax.experimental.pallas import tpu as pltpu
from jax.experimental.pallas import tpu_sc as plsc

_NCLS = 256
_NF = 256
_NPIX = 16384
_PPT = 1024
_P1 = 128
_C1 = 128
_G1 = _P1 // 16
_PC = 1024
_FPT = 16
_FH = 8


def _sc_body(lg_hbm, emb_hbm,
             sem_hbm, sum_hbm, sq_hbm, cnt_hbm,
             lg_v, emb_v, semt_v, semc_v,
             bins_s, bins_q, cnt_bins,
             comp_s, comp_q, cnt_c):
    c = lax.axis_index("c")
    s = lax.axis_index("s")
    lane = lax.iota(jnp.int32, 16)
    zf = jnp.zeros((16,), jnp.float32)
    onef = jnp.ones((16,), jnp.float32)

    def _z0(i, _):
        cnt_bins[pl.ds(i * 16, 16)] = zf
        return 0
    lax.fori_loop(0, _NCLS * 16 // 16, _z0, 0)

    base_pix = s * _PPT

    def chunk_body(k, _):
        p0 = base_pix + k * _P1

        bv = tuple(jnp.full((16,), -jnp.inf, jnp.float32) for _ in range(_G1))
        bi = tuple(jnp.zeros((16,), jnp.int32) for _ in range(_G1))

        for cc in range(_NCLS // _C1):
            pltpu.sync_copy(lg_hbm.at[c, pl.ds(cc * _C1, _C1),
                                      pl.ds(p0, _P1)], lg_v)

            def cls_body(cls, carry):
                pbv, pbi = carry
                civ = jnp.full((16,), cc * _C1, jnp.int32) + cls
                nbv, nbi = [], []
                for g in range(_G1):
                    v = lg_v[cls, pl.ds(g * 16, 16)]
                    m = v > pbv[g]
                    nbv.append(jnp.where(m, v, pbv[g]))
                    nbi.append(jnp.where(m, civ, pbi[g]))
                return tuple(nbv), tuple(nbi)

            bv, bi = lax.fori_loop(0, _C1, cls_body, (bv, bi))
        for g in range(_G1):
            semt_v[pl.ds(k * _P1 + g * 16, 16)] = bi[g]
            plsc.addupdate_scatter(cnt_bins, [bi[g] * 16 + lane], onef)
        return 0

    lax.fori_loop(0, _PPT // _P1, chunk_body, 0)

    pltpu.sync_copy(semt_v, sem_hbm.at[pl.ds(c * _NPIX + s * _PPT, _PPT)])

    def cnt_red(cb, _):
        cidx = (jnp.full((16,), 0, jnp.int32) + cb * 16 + lane) * 16
        acc = zf
        for k in range(16):
            acc = acc + plsc.load_gather(cnt_bins, [cidx + ((k + lane) & 15)])
        cnt_c[pl.ds(cb * 16, 16)] = acc
        return 0
    lax.fori_loop(0, _NCLS // 16, cnt_red, 0)
    pltpu.sync_copy(cnt_c, cnt_hbm.at[pl.ds((c * 16 + s) * _NCLS, _NCLS)])

    plsc.subcore_barrier()

    for ph in range(2):
        fbase = s * _FPT + ph * _FH

        def _z1(i, _):
            for u in range(4):
                bins_s[pl.ds((i * 4 + u) * 16, 16)] = zf
                bins_q[pl.ds((i * 4 + u) * 16, 16)] = zf
            return 0
        lax.fori_loop(0, _NCLS * _FH // 4, _z1, 0)

        def pix_body(pc, _):
            p0 = pc * _PC
            pltpu.sync_copy(sem_hbm.at[pl.ds(c * _NPIX + p0, _PC)], semc_v)
            pltpu.sync_copy(emb_hbm.at[c, pl.ds(fbase, _FH), pl.ds(p0, _PC)],
                            emb_v)

            def g_body(g, _):
                idx = semc_v[pl.ds(g * 16, 16)]
                base = idx * (_FH * 16)
                for f in range(_FH):
                    v = emb_v[f, pl.ds(g * 16, 16)]
                    fidx = base + (f * 16) + ((lane + f) & 15)
                    plsc.addupdate_scatter(bins_s, [fidx], v)
                    plsc.addupdate_scatter(bins_q, [fidx], v * v)
                return 0

            lax.fori_loop(0, _PC // 16, g_body, 0)
            return 0

        lax.fori_loop(0, _NPIX // _PC, pix_body, 0)

        def red_body(cb, _):
            base = cb * (2 * _FH * 16)
            acc_s = zf
            acc_q = zf
            for k in range(16):
                pk = ((lane // 8) * (_FH * 16) + (lane % 8) * 16
                      + ((k + (lane % 8) + 8 * (lane // 8)) & 15))
                acc_s = acc_s + plsc.load_gather(bins_s, [base + pk])
                acc_q = acc_q + plsc.load_gather(bins_q, [base + pk])
            fi = ph * _FH + lane % 8
            ci = cb * 2 + lane // 8
            plsc.store_scatter(comp_s, [fi, ci], acc_s)
            plsc.store_scatter(comp_q, [fi, ci], acc_q)
            return 0

        lax.fori_loop(0, _NCLS // 2, red_body, 0)

    pltpu.sync_copy(comp_s, sum_hbm.at[c, pl.ds(s * _FPT, _FPT), :])
    pltpu.sync_copy(comp_q, sq_hbm.at[c, pl.ds(s * _FPT, _FPT), :])


def _epi_body(it_ref, s_ref, q_ref, cnt_ref, feat_ref, ex_ref, ex2_ref,
              cntin_ref, nf_ref, nex_ref, nex2_ref, ncnt_ref):
    train = (it_ref[0] != 0).astype(jnp.float32)
    S = jnp.transpose(s_ref[0] + s_ref[1])
    Q = jnp.transpose(q_ref[0] + q_ref[1])
    cnew_row = jnp.sum(cnt_ref[...], axis=0, keepdims=True)
    cold_row = cntin_ref[...]
    ones_row = jnp.ones((1, _NF), jnp.float32)
    dn = (((0,), (0,)), ((), ()))
    cnew = lax.dot_general(cnew_row, ones_row, dn,
                           preferred_element_type=jnp.float32)
    cold = lax.dot_general(cold_row, ones_row, dn,
                           preferred_element_type=jnp.float32)
    feat = feat_ref[...]
    upd_f = (feat * cold + S) / (cold + cnew + 1e-8)
    nf_ref[...] = train * upd_f + (1.0 - train) * feat
    nex_ref[...] = ex_ref[...] + train * S
    nex2_ref[...] = ex2_ref[...] + train * Q
    ncnt_ref[...] = cold_row + train * cnew_row


def kernel(unified_embedding, logits, gt, is_train, dataset_ids, features, ex,
           ex2, count):
    B, C, H, W = unified_embedding.shape
    L = logits.shape[1]
    N = H * W
    emb = unified_embedding.reshape(B, C, N)
    lg = logits.reshape(B, L, N)

    mesh = plsc.VectorSubcoreMesh(core_axis_name="c", subcore_axis_name="s")
    sc = pl.kernel(
        _sc_body,
        out_type=[
            jax.ShapeDtypeStruct((B * N,), jnp.int32),
            jax.ShapeDtypeStruct((B, C, L), jnp.float32),
            jax.ShapeDtypeStruct((B, C, L), jnp.float32),
            jax.ShapeDtypeStruct((B * 16 * L,), jnp.float32),
        ],
        mesh=mesh,
        compiler_params=pltpu.CompilerParams(needs_layout_passes=False),
        scratch_types=[
            pltpu.VMEM((_C1, _P1), jnp.float32),
            pltpu.VMEM((_FH, _PC), jnp.float32),
            pltpu.VMEM((_PPT,), jnp.int32),
            pltpu.VMEM((_PC,), jnp.int32),
            pltpu.VMEM((_NCLS * _FH * 16,), jnp.float32),
            pltpu.VMEM((_NCLS * _FH * 16,), jnp.float32),
            pltpu.VMEM((_NCLS * 16,), jnp.float32),
            pltpu.VMEM((_FPT, _NCLS), jnp.float32),
            pltpu.VMEM((_FPT, _NCLS), jnp.float32),
            pltpu.VMEM((_NCLS,), jnp.float32),
        ],
    )
    _sem, sums, sqs, cnts = sc(lg, emb)

    it = jnp.asarray(is_train, jnp.int32).reshape(1)
    out = pl.pallas_call(
        _epi_body,
        in_specs=[
            pl.BlockSpec(memory_space=pltpu.SMEM),
            pl.BlockSpec((B, C, L), lambda: (0, 0, 0)),
            pl.BlockSpec((B, C, L), lambda: (0, 0, 0)),
            pl.BlockSpec((B * 16, L), lambda: (0, 0)),
            pl.BlockSpec((L, C), lambda: (0, 0)),
            pl.BlockSpec((L, C), lambda: (0, 0)),
            pl.BlockSpec((L, C), lambda: (0, 0)),
            pl.BlockSpec((1, L), lambda: (0, 0)),
        ],
        out_specs=[
            pl.BlockSpec((L, C), lambda: (0, 0)),
            pl.BlockSpec((L, C), lambda: (0, 0)),
            pl.BlockSpec((L, C), lambda: (0, 0)),
            pl.BlockSpec((1, L), lambda: (0, 0)),
        ],
        out_shape=[
            jax.ShapeDtypeStruct((L, C), jnp.float32),
            jax.ShapeDtypeStruct((L, C), jnp.float32),
            jax.ShapeDtypeStruct((L, C), jnp.float32),
            jax.ShapeDtypeStruct((1, L), jnp.float32),
        ],
    )(it, sums, sqs, cnts.reshape(B * 16, L), features, ex, ex2,
      count.reshape(1, L))

    new_features, new_ex, new_ex2, new_count = out
    acc_loss = jnp.zeros((), jnp.float32)
    return (acc_loss, new_features, new_ex, new_ex2, new_count.reshape(L))


# speedup vs baseline: 1.1954x; 1.1476x over previous
import jax
import jax.numpy as jnp
from jax import lax
from jax.experimental import pallas as pl
from jax.experimental.pallas import tpu as pltpu
from jax.experimental.pallas import tpu_sc as plsc

_NCLS = 256
_NF = 256
_NPIX = 16384
_PPT = 1024
_P1 = 128
_C1 = 128
_G1 = _P1 // 16
_PC = 1024
_FPT = 16
_FH = 8


def _sc_body(lg_hbm, emb_hbm,
             sem_hbm, sum_hbm, sq_hbm, cnt_hbm,
             lg_v, emb_v, semt_v, semc_v,
             bins_s, bins_q, cnt_bins,
             comp_s, comp_q, cnt_c):
    c = lax.axis_index("c")
    s = lax.axis_index("s")
    lane = lax.iota(jnp.int32, 16)
    zf = jnp.zeros((16,), jnp.float32)
    onef = jnp.ones((16,), jnp.float32)

    def _z0(i, _):
        cnt_bins[pl.ds(i * 16, 16)] = zf
        return 0
    lax.fori_loop(0, _NCLS * 16 // 16, _z0, 0)

    base_pix = s * _PPT

    def chunk_body(k, _):
        p0 = base_pix + k * _P1

        bv = tuple(jnp.full((16,), -jnp.inf, jnp.float32) for _ in range(_G1))
        bi = tuple(jnp.zeros((16,), jnp.int32) for _ in range(_G1))

        for cc in range(_NCLS // _C1):
            pltpu.sync_copy(lg_hbm.at[c, pl.ds(cc * _C1, _C1),
                                      pl.ds(p0, _P1)], lg_v)

            def cls_body(cls, carry):
                pbv, pbi = carry
                civ = jnp.full((16,), cc * _C1, jnp.int32) + cls
                nbv, nbi = [], []
                for g in range(_G1):
                    v = lg_v[cls, pl.ds(g * 16, 16)]
                    m = v > pbv[g]
                    nbv.append(jnp.where(m, v, pbv[g]))
                    nbi.append(jnp.where(m, civ, pbi[g]))
                return tuple(nbv), tuple(nbi)

            bv, bi = lax.fori_loop(0, _C1, cls_body, (bv, bi))
        for g in range(_G1):
            semt_v[pl.ds(k * _P1 + g * 16, 16)] = bi[g]
            plsc.addupdate_scatter(cnt_bins, [bi[g] * 16 + lane], onef)
        return 0

    lax.fori_loop(0, _PPT // _P1, chunk_body, 0)

    pltpu.sync_copy(semt_v, sem_hbm.at[pl.ds(c * _NPIX + s * _PPT, _PPT)])

    def cnt_red(cb, _):
        cidx = (jnp.full((16,), 0, jnp.int32) + cb * 16 + lane) * 16
        acc = zf
        for k in range(16):
            acc = acc + plsc.load_gather(cnt_bins, [cidx + ((k + lane) & 15)])
        cnt_c[pl.ds(cb * 16, 16)] = acc
        return 0
    lax.fori_loop(0, _NCLS // 16, cnt_red, 0)
    pltpu.sync_copy(cnt_c, cnt_hbm.at[pl.ds((c * 16 + s) * _NCLS, _NCLS)])

    plsc.subcore_barrier()

    for ph in range(2):
        fbase = s * _FPT + ph * _FH

        def _z1(i, _):
            for u in range(4):
                bins_s[pl.ds((i * 4 + u) * 16, 16)] = zf
                bins_q[pl.ds((i * 4 + u) * 16, 16)] = zf
            return 0
        lax.fori_loop(0, _NCLS * _FH // 4, _z1, 0)

        def pix_body(pc, _):
            p0 = pc * _PC
            pltpu.sync_copy(sem_hbm.at[pl.ds(c * _NPIX + p0, _PC)], semc_v)
            pltpu.sync_copy(emb_hbm.at[c, pl.ds(fbase, _FH), pl.ds(p0, _PC)],
                            emb_v)

            def g_body(g2, _):
                idx0 = semc_v[pl.ds(g2 * 32, 16)]
                idx1 = semc_v[pl.ds(g2 * 32 + 16, 16)]
                base0 = idx0 * (_FH * 16)
                base1 = idx1 * (_FH * 16)
                for f in range(_FH):
                    sl = (lane + f) & 15
                    v0 = emb_v[f, pl.ds(g2 * 32, 16)]
                    v1 = emb_v[f, pl.ds(g2 * 32 + 16, 16)]
                    fidx0 = base0 + (f * 16) + sl
                    fidx1 = base1 + (f * 16) + sl
                    plsc.addupdate_scatter(bins_s, [fidx0], v0)
                    plsc.addupdate_scatter(bins_s, [fidx1], v1)
                    plsc.addupdate_scatter(bins_q, [fidx0], v0 * v0)
                    plsc.addupdate_scatter(bins_q, [fidx1], v1 * v1)
                return 0

            lax.fori_loop(0, _PC // 32, g_body, 0)
            return 0

        lax.fori_loop(0, _NPIX // _PC, pix_body, 0)

        def red_body(cb, _):
            base = cb * (2 * _FH * 16)
            acc_s = zf
            acc_q = zf
            for k in range(16):
                pk = ((lane // 8) * (_FH * 16) + (lane % 8) * 16
                      + ((k + (lane % 8) + 8 * (lane // 8)) & 15))
                acc_s = acc_s + plsc.load_gather(bins_s, [base + pk])
                acc_q = acc_q + plsc.load_gather(bins_q, [base + pk])
            fi = ph * _FH + lane % 8
            ci = cb * 2 + lane // 8
            plsc.store_scatter(comp_s, [fi, ci], acc_s)
            plsc.store_scatter(comp_q, [fi, ci], acc_q)
            return 0

        lax.fori_loop(0, _NCLS // 2, red_body, 0)

    pltpu.sync_copy(comp_s, sum_hbm.at[c, pl.ds(s * _FPT, _FPT), :])
    pltpu.sync_copy(comp_q, sq_hbm.at[c, pl.ds(s * _FPT, _FPT), :])


def _epi_body(it_ref, s_ref, q_ref, cnt_ref, feat_ref, ex_ref, ex2_ref,
              cntin_ref, nf_ref, nex_ref, nex2_ref, ncnt_ref):
    train = (it_ref[0] != 0).astype(jnp.float32)
    S = jnp.transpose(s_ref[0] + s_ref[1])
    Q = jnp.transpose(q_ref[0] + q_ref[1])
    cnew_row = jnp.sum(cnt_ref[...], axis=0, keepdims=True)
    cold_row = cntin_ref[...]
    ones_row = jnp.ones((1, _NF), jnp.float32)
    dn = (((0,), (0,)), ((), ()))
    cnew = lax.dot_general(cnew_row, ones_row, dn,
                           preferred_element_type=jnp.float32)
    cold = lax.dot_general(cold_row, ones_row, dn,
                           preferred_element_type=jnp.float32)
    feat = feat_ref[...]
    upd_f = (feat * cold + S) / (cold + cnew + 1e-8)
    nf_ref[...] = train * upd_f + (1.0 - train) * feat
    nex_ref[...] = ex_ref[...] + train * S
    nex2_ref[...] = ex2_ref[...] + train * Q
    ncnt_ref[...] = cold_row + train * cnew_row


def kernel(unified_embedding, logits, gt, is_train, dataset_ids, features, ex,
           ex2, count):
    B, C, H, W = unified_embedding.shape
    L = logits.shape[1]
    N = H * W
    emb = unified_embedding.reshape(B, C, N)
    lg = logits.reshape(B, L, N)

    mesh = plsc.VectorSubcoreMesh(core_axis_name="c", subcore_axis_name="s")
    sc = pl.kernel(
        _sc_body,
        out_type=[
            jax.ShapeDtypeStruct((B * N,), jnp.int32),
            jax.ShapeDtypeStruct((B, C, L), jnp.float32),
            jax.ShapeDtypeStruct((B, C, L), jnp.float32),
            jax.ShapeDtypeStruct((B * 16 * L,), jnp.float32),
        ],
        mesh=mesh,
        compiler_params=pltpu.CompilerParams(needs_layout_passes=False),
        scratch_types=[
            pltpu.VMEM((_C1, _P1), jnp.float32),
            pltpu.VMEM((_FH, _PC), jnp.float32),
            pltpu.VMEM((_PPT,), jnp.int32),
            pltpu.VMEM((_PC,), jnp.int32),
            pltpu.VMEM((_NCLS * _FH * 16,), jnp.float32),
            pltpu.VMEM((_NCLS * _FH * 16,), jnp.float32),
            pltpu.VMEM((_NCLS * 16,), jnp.float32),
            pltpu.VMEM((_FPT, _NCLS), jnp.float32),
            pltpu.VMEM((_FPT, _NCLS), jnp.float32),
            pltpu.VMEM((_NCLS,), jnp.float32),
        ],
    )
    _sem, sums, sqs, cnts = sc(lg, emb)

    it = jnp.asarray(is_train, jnp.int32).reshape(1)
    out = pl.pallas_call(
        _epi_body,
        in_specs=[
            pl.BlockSpec(memory_space=pltpu.SMEM),
            pl.BlockSpec((B, C, L), lambda: (0, 0, 0)),
            pl.BlockSpec((B, C, L), lambda: (0, 0, 0)),
            pl.BlockSpec((B * 16, L), lambda: (0, 0)),
            pl.BlockSpec((L, C), lambda: (0, 0)),
            pl.BlockSpec((L, C), lambda: (0, 0)),
            pl.BlockSpec((L, C), lambda: (0, 0)),
            pl.BlockSpec((1, L), lambda: (0, 0)),
        ],
        out_specs=[
            pl.BlockSpec((L, C), lambda: (0, 0)),
            pl.BlockSpec((L, C), lambda: (0, 0)),
            pl.BlockSpec((L, C), lambda: (0, 0)),
            pl.BlockSpec((1, L), lambda: (0, 0)),
        ],
        out_shape=[
            jax.ShapeDtypeStruct((L, C), jnp.float32),
            jax.ShapeDtypeStruct((L, C), jnp.float32),
            jax.ShapeDtypeStruct((L, C), jnp.float32),
            jax.ShapeDtypeStruct((1, L), jnp.float32),
        ],
    )(it, sums, sqs, cnts.reshape(B * 16, L), features, ex, ex2,
      count.reshape(1, L))

    new_features, new_ex, new_ex2, new_count = out
    acc_loss = jnp.zeros((), jnp.float32)
    return (acc_loss, new_features, new_ex, new_ex2, new_count.reshape(L))


# speedup vs baseline: 1.5996x; 1.3380x over previous
import jax
import jax.numpy as jnp
from jax import lax
from jax.experimental import pallas as pl
from jax.experimental.pallas import tpu as pltpu
from jax.experimental.pallas import tpu_sc as plsc

_NCLS = 256
_NF = 256
_H = 128
_W = 128
_NPIX = _H * _W
_PPT = 1024
_RPT = _PPT // _W
_C1 = 32
_PC = 1024
_RC = _PC // _W
_FPT = 16
_FH = 8


def _sc_body(lg_hbm, emb_hbm,
             sem_hbm, sum_hbm, sq_hbm, cnt_hbm,
             lg_v, emb_v, semc_v, bvt, bit,
             bins_s, bins_q, cnt_bins,
             comp_s, comp_q, cnt_c):
    c = lax.axis_index("c")
    s = lax.axis_index("s")
    lane = lax.iota(jnp.int32, 16)
    zf = jnp.zeros((16,), jnp.float32)
    onef = jnp.ones((16,), jnp.float32)

    def _z0(i, _):
        cnt_bins[pl.ds(i * 16, 16)] = zf
        return 0
    lax.fori_loop(0, _NCLS * 16 // 16, _z0, 0)

    h0 = s * _RPT

    for cc in range(_NCLS // _C1):
        pltpu.sync_copy(
            lg_hbm.at[pl.ds(c * _NCLS + cc * _C1, _C1), pl.ds(h0, _RPT), :],
            lg_v)

        def grp_body(g, _):
            r = g // 8
            col = (g % 8) * 16
            if cc == 0:
                bv = jnp.full((16,), -jnp.inf, jnp.float32)
                bi = jnp.zeros((16,), jnp.int32)
            else:
                bv = bvt[pl.ds(g * 16, 16)]
                bi = bit[pl.ds(g * 16, 16)]
            for cls in range(_C1):
                v = lg_v[cls, r, pl.ds(col, 16)]
                m = v > bv
                bv = jnp.where(m, v, bv)
                bi = jnp.where(m, jnp.full((16,), cc * _C1 + cls, jnp.int32),
                               bi)
            bvt[pl.ds(g * 16, 16)] = bv
            bit[pl.ds(g * 16, 16)] = bi
            return 0

        lax.fori_loop(0, _PPT // 16, grp_body, 0)

    def hist_body(g, _):
        bi = bit[pl.ds(g * 16, 16)]
        plsc.addupdate_scatter(cnt_bins, [bi * 16 + lane], onef)
        return 0
    lax.fori_loop(0, _PPT // 16, hist_body, 0)
    pltpu.sync_copy(bit, sem_hbm.at[pl.ds(c * _NPIX + s * _PPT, _PPT)])

    def cnt_red(cb, _):
        cidx = (jnp.full((16,), 0, jnp.int32) + cb * 16 + lane) * 16
        acc = zf
        for k in range(16):
            acc = acc + plsc.load_gather(cnt_bins, [cidx + ((k + lane) & 15)])
        cnt_c[pl.ds(cb * 16, 16)] = acc
        return 0
    lax.fori_loop(0, _NCLS // 16, cnt_red, 0)
    pltpu.sync_copy(cnt_c, cnt_hbm.at[pl.ds((c * 16 + s) * _NCLS, _NCLS)])

    plsc.subcore_barrier()

    for ph in range(2):
        fbase = s * _FPT + ph * _FH

        def _z1(i, _):
            for u in range(4):
                bins_s[pl.ds((i * 4 + u) * 16, 16)] = zf
                bins_q[pl.ds((i * 4 + u) * 16, 16)] = zf
            return 0
        lax.fori_loop(0, _NCLS * _FH // 4, _z1, 0)

        def pix_body(pc, _):
            p0 = pc * _PC
            pltpu.sync_copy(sem_hbm.at[pl.ds(c * _NPIX + p0, _PC)], semc_v)
            pltpu.sync_copy(
                emb_hbm.at[pl.ds(c * _NF + fbase, _FH),
                           pl.ds(pc * _RC, _RC), :],
                emb_v)

            def g_body(g2, _):
                r0 = (g2 * 2) // 8
                col0 = ((g2 * 2) % 8) * 16
                r1 = (g2 * 2 + 1) // 8
                col1 = ((g2 * 2 + 1) % 8) * 16
                idx0 = semc_v[pl.ds(g2 * 32, 16)]
                idx1 = semc_v[pl.ds(g2 * 32 + 16, 16)]
                base0 = idx0 * (_FH * 16)
                base1 = idx1 * (_FH * 16)
                for f in range(_FH):
                    sl = (lane + f) & 15
                    v0 = emb_v[f, r0, pl.ds(col0, 16)]
                    v1 = emb_v[f, r1, pl.ds(col1, 16)]
                    fidx0 = base0 + (f * 16) + sl
                    fidx1 = base1 + (f * 16) + sl
                    plsc.addupdate_scatter(bins_s, [fidx0], v0)
                    plsc.addupdate_scatter(bins_s, [fidx1], v1)
                    plsc.addupdate_scatter(bins_q, [fidx0], v0 * v0)
                    plsc.addupdate_scatter(bins_q, [fidx1], v1 * v1)
                return 0

            lax.fori_loop(0, _PC // 32, g_body, 0)
            return 0

        lax.fori_loop(0, _NPIX // _PC, pix_body, 0)

        def red_body(cb, _):
            base = cb * (2 * _FH * 16)
            acc_s = zf
            acc_q = zf
            for k in range(16):
                pk = ((lane // 8) * (_FH * 16) + (lane % 8) * 16
                      + ((k + (lane % 8) + 8 * (lane // 8)) & 15))
                acc_s = acc_s + plsc.load_gather(bins_s, [base + pk])
                acc_q = acc_q + plsc.load_gather(bins_q, [base + pk])
            fi = ph * _FH + lane % 8
            ci = cb * 2 + lane // 8
            plsc.store_scatter(comp_s, [fi, ci], acc_s)
            plsc.store_scatter(comp_q, [fi, ci], acc_q)
            return 0

        lax.fori_loop(0, _NCLS // 2, red_body, 0)

    pltpu.sync_copy(comp_s, sum_hbm.at[c, pl.ds(s * _FPT, _FPT), :])
    pltpu.sync_copy(comp_q, sq_hbm.at[c, pl.ds(s * _FPT, _FPT), :])


def _epi_body(it_ref, s_ref, q_ref, cnt_ref, feat_ref, ex_ref, ex2_ref,
              cntin_ref, nf_ref, nex_ref, nex2_ref, ncnt_ref):
    train = (it_ref[0] != 0).astype(jnp.float32)
    S = jnp.transpose(s_ref[0] + s_ref[1])
    Q = jnp.transpose(q_ref[0] + q_ref[1])
    cnew_row = jnp.sum(cnt_ref[...], axis=0, keepdims=True)
    cold_row = cntin_ref[...]
    ones_row = jnp.ones((1, _NF), jnp.float32)
    dn = (((0,), (0,)), ((), ()))
    cnew = lax.dot_general(cnew_row, ones_row, dn,
                           preferred_element_type=jnp.float32)
    cold = lax.dot_general(cold_row, ones_row, dn,
                           preferred_element_type=jnp.float32)
    feat = feat_ref[...]
    upd_f = (feat * cold + S) / (cold + cnew + 1e-8)
    nf_ref[...] = train * upd_f + (1.0 - train) * feat
    nex_ref[...] = ex_ref[...] + train * S
    nex2_ref[...] = ex2_ref[...] + train * Q
    ncnt_ref[...] = cold_row + train * cnew_row


def kernel(unified_embedding, logits, gt, is_train, dataset_ids, features, ex,
           ex2, count):
    B, C, H, W = unified_embedding.shape
    L = logits.shape[1]
    N = H * W
    emb = unified_embedding.reshape(B * C, H, W)
    lg = logits.reshape(B * L, H, W)

    mesh = plsc.VectorSubcoreMesh(core_axis_name="c", subcore_axis_name="s")
    sc = pl.kernel(
        _sc_body,
        out_type=[
            jax.ShapeDtypeStruct((B * N,), jnp.int32),
            jax.ShapeDtypeStruct((B, C, L), jnp.float32),
            jax.ShapeDtypeStruct((B, C, L), jnp.float32),
            jax.ShapeDtypeStruct((B * 16 * L,), jnp.float32),
        ],
        mesh=mesh,
        compiler_params=pltpu.CompilerParams(needs_layout_passes=False),
        scratch_types=[
            pltpu.VMEM((_C1, _RPT, _W), jnp.float32),
            pltpu.VMEM((_FH, _RC, _W), jnp.float32),
            pltpu.VMEM((_PC,), jnp.int32),
            pltpu.VMEM((_PPT,), jnp.float32),
            pltpu.VMEM((_PPT,), jnp.int32),
            pltpu.VMEM((_NCLS * _FH * 16,), jnp.float32),
            pltpu.VMEM((_NCLS * _FH * 16,), jnp.float32),
            pltpu.VMEM((_NCLS * 16,), jnp.float32),
            pltpu.VMEM((_FPT, _NCLS), jnp.float32),
            pltpu.VMEM((_FPT, _NCLS), jnp.float32),
            pltpu.VMEM((_NCLS,), jnp.float32),
        ],
    )
    _sem, sums, sqs, cnts = sc(lg, emb)

    it = jnp.asarray(is_train, jnp.int32).reshape(1)
    out = pl.pallas_call(
        _epi_body,
        in_specs=[
            pl.BlockSpec(memory_space=pltpu.SMEM),
            pl.BlockSpec((B, C, L), lambda: (0, 0, 0)),
            pl.BlockSpec((B, C, L), lambda: (0, 0, 0)),
            pl.BlockSpec((B * 16, L), lambda: (0, 0)),
            pl.BlockSpec((L, C), lambda: (0, 0)),
            pl.BlockSpec((L, C), lambda: (0, 0)),
            pl.BlockSpec((L, C), lambda: (0, 0)),
            pl.BlockSpec((1, L), lambda: (0, 0)),
        ],
        out_specs=[
            pl.BlockSpec((L, C), lambda: (0, 0)),
            pl.BlockSpec((L, C), lambda: (0, 0)),
            pl.BlockSpec((L, C), lambda: (0, 0)),
            pl.BlockSpec((1, L), lambda: (0, 0)),
        ],
        out_shape=[
            jax.ShapeDtypeStruct((L, C), jnp.float32),
            jax.ShapeDtypeStruct((L, C), jnp.float32),
            jax.ShapeDtypeStruct((L, C), jnp.float32),
            jax.ShapeDtypeStruct((1, L), jnp.float32),
        ],
    )(it, sums, sqs, cnts.reshape(B * 16, L), features, ex, ex2,
      count.reshape(1, L))

    new_features, new_ex, new_ex2, new_count = out
    acc_loss = jnp.zeros((), jnp.float32)
    return (acc_loss, new_features, new_ex, new_ex2, new_count.reshape(L))


# speedup vs baseline: 1.7036x; 1.0650x over previous
import jax
import jax.numpy as jnp
from jax import lax
from jax.experimental import pallas as pl
from jax.experimental.pallas import tpu as pltpu
from jax.experimental.pallas import tpu_sc as plsc

_NCLS = 256
_NF = 256
_H = 128
_W = 128
_NPIX = _H * _W
_PPT = 1024
_RPT = _PPT // _W
_C1 = 32
_PC = 1024
_RC = _PC // _W
_FPT = 16
_FH = 8


def _sc_body(lg_hbm, emb_hbm,
             sem_hbm, sum_hbm, sq_hbm, cnt_hbm,
             lg_v, emb_v, semc_v, bvt, bit,
             bins_s, bins_q, cnt_bins,
             comp_s, comp_q, cnt_c):
    c = lax.axis_index("c")
    s = lax.axis_index("s")
    lane = lax.iota(jnp.int32, 16)
    zf = jnp.zeros((16,), jnp.float32)
    onef = jnp.ones((16,), jnp.float32)

    def _z0(i, _):
        cnt_bins[pl.ds(i * 16, 16)] = zf
        return 0
    lax.fori_loop(0, _NCLS * 16 // 16, _z0, 0)

    h0 = s * _RPT

    for cc in range(_NCLS // _C1):
        pltpu.sync_copy(
            lg_hbm.at[pl.ds(c * _NCLS + cc * _C1, _C1), pl.ds(h0, _RPT), :],
            lg_v)

        def grp_body(g4, _):
            r = g4 // 2
            colb = (g4 % 2) * 64
            if cc == 0:
                bv = [jnp.full((16,), -jnp.inf, jnp.float32)
                      for _ in range(4)]
                bi = [jnp.zeros((16,), jnp.int32) for _ in range(4)]
            else:
                bv = [bvt[pl.ds(g4 * 64 + u * 16, 16)] for u in range(4)]
                bi = [bit[pl.ds(g4 * 64 + u * 16, 16)] for u in range(4)]
            for cls in range(_C1):
                civ = jnp.full((16,), cc * _C1 + cls, jnp.int32)
                for u in range(4):
                    v = lg_v[cls, r, pl.ds(colb + u * 16, 16)]
                    m = v > bv[u]
                    bv[u] = jnp.where(m, v, bv[u])
                    bi[u] = jnp.where(m, civ, bi[u])
            for u in range(4):
                bvt[pl.ds(g4 * 64 + u * 16, 16)] = bv[u]
                bit[pl.ds(g4 * 64 + u * 16, 16)] = bi[u]
            return 0

        lax.fori_loop(0, _PPT // 64, grp_body, 0)

    def hist_body(g, _):
        bi = bit[pl.ds(g * 16, 16)]
        plsc.addupdate_scatter(cnt_bins, [bi * 16 + lane], onef)
        return 0
    lax.fori_loop(0, _PPT // 16, hist_body, 0)
    pltpu.sync_copy(bit, sem_hbm.at[pl.ds(c * _NPIX + s * _PPT, _PPT)])

    def cnt_red(cb, _):
        cidx = (jnp.full((16,), 0, jnp.int32) + cb * 16 + lane) * 16
        acc = zf
        for k in range(16):
            acc = acc + plsc.load_gather(cnt_bins, [cidx + ((k + lane) & 15)])
        cnt_c[pl.ds(cb * 16, 16)] = acc
        return 0
    lax.fori_loop(0, _NCLS // 16, cnt_red, 0)
    pltpu.sync_copy(cnt_c, cnt_hbm.at[pl.ds((c * 16 + s) * _NCLS, _NCLS)])

    plsc.subcore_barrier()

    for ph in range(2):
        fbase = s * _FPT + ph * _FH

        def _z1(i, _):
            for u in range(4):
                bins_s[pl.ds((i * 4 + u) * 16, 16)] = zf
                bins_q[pl.ds((i * 4 + u) * 16, 16)] = zf
            return 0
        lax.fori_loop(0, _NCLS * _FH // 4, _z1, 0)

        def pix_body(pc, _):
            p0 = pc * _PC
            pltpu.sync_copy(sem_hbm.at[pl.ds(c * _NPIX + p0, _PC)], semc_v)
            pltpu.sync_copy(
                emb_hbm.at[pl.ds(c * _NF + fbase, _FH),
                           pl.ds(pc * _RC, _RC), :],
                emb_v)

            def g_body(g2, _):
                r0 = (g2 * 2) // 8
                col0 = ((g2 * 2) % 8) * 16
                r1 = (g2 * 2 + 1) // 8
                col1 = ((g2 * 2 + 1) % 8) * 16
                idx0 = semc_v[pl.ds(g2 * 32, 16)]
                idx1 = semc_v[pl.ds(g2 * 32 + 16, 16)]
                base0 = idx0 * (_FH * 16)
                base1 = idx1 * (_FH * 16)
                for f in range(_FH):
                    sl = (lane + f) & 15
                    v0 = emb_v[f, r0, pl.ds(col0, 16)]
                    v1 = emb_v[f, r1, pl.ds(col1, 16)]
                    fidx0 = base0 + (f * 16) + sl
                    fidx1 = base1 + (f * 16) + sl
                    plsc.addupdate_scatter(bins_s, [fidx0], v0)
                    plsc.addupdate_scatter(bins_s, [fidx1], v1)
                    plsc.addupdate_scatter(bins_q, [fidx0], v0 * v0)
                    plsc.addupdate_scatter(bins_q, [fidx1], v1 * v1)
                return 0

            lax.fori_loop(0, _PC // 32, g_body, 0)
            return 0

        lax.fori_loop(0, _NPIX // _PC, pix_body, 0)

        def red_body(cb, _):
            base = cb * (2 * _FH * 16)
            acc_s = zf
            acc_q = zf
            for k in range(16):
                pk = ((lane // 8) * (_FH * 16) + (lane % 8) * 16
                      + ((k + (lane % 8) + 8 * (lane // 8)) & 15))
                acc_s = acc_s + plsc.load_gather(bins_s, [base + pk])
                acc_q = acc_q + plsc.load_gather(bins_q, [base + pk])
            fi = ph * _FH + lane % 8
            ci = cb * 2 + lane // 8
            plsc.store_scatter(comp_s, [fi, ci], acc_s)
            plsc.store_scatter(comp_q, [fi, ci], acc_q)
            return 0

        lax.fori_loop(0, _NCLS // 2, red_body, 0)

    pltpu.sync_copy(comp_s, sum_hbm.at[c, pl.ds(s * _FPT, _FPT), :])
    pltpu.sync_copy(comp_q, sq_hbm.at[c, pl.ds(s * _FPT, _FPT), :])


def _epi_body(it_ref, s_ref, q_ref, cnt_ref, feat_ref, ex_ref, ex2_ref,
              cntin_ref, nf_ref, nex_ref, nex2_ref, ncnt_ref):
    train = (it_ref[0] != 0).astype(jnp.float32)
    S = jnp.transpose(s_ref[0] + s_ref[1])
    Q = jnp.transpose(q_ref[0] + q_ref[1])
    cnew_row = jnp.sum(cnt_ref[...], axis=0, keepdims=True)
    cold_row = cntin_ref[...]
    ones_row = jnp.ones((1, _NF), jnp.float32)
    dn = (((0,), (0,)), ((), ()))
    cnew = lax.dot_general(cnew_row, ones_row, dn,
                           preferred_element_type=jnp.float32)
    cold = lax.dot_general(cold_row, ones_row, dn,
                           preferred_element_type=jnp.float32)
    feat = feat_ref[...]
    upd_f = (feat * cold + S) / (cold + cnew + 1e-8)
    nf_ref[...] = train * upd_f + (1.0 - train) * feat
    nex_ref[...] = ex_ref[...] + train * S
    nex2_ref[...] = ex2_ref[...] + train * Q
    ncnt_ref[...] = cold_row + train * cnew_row


def kernel(unified_embedding, logits, gt, is_train, dataset_ids, features, ex,
           ex2, count):
    B, C, H, W = unified_embedding.shape
    L = logits.shape[1]
    N = H * W
    emb = unified_embedding.reshape(B * C, H, W)
    lg = logits.reshape(B * L, H, W)

    mesh = plsc.VectorSubcoreMesh(core_axis_name="c", subcore_axis_name="s")
    sc = pl.kernel(
        _sc_body,
        out_type=[
            jax.ShapeDtypeStruct((B * N,), jnp.int32),
            jax.ShapeDtypeStruct((B, C, L), jnp.float32),
            jax.ShapeDtypeStruct((B, C, L), jnp.float32),
            jax.ShapeDtypeStruct((B * 16 * L,), jnp.float32),
        ],
        mesh=mesh,
        compiler_params=pltpu.CompilerParams(needs_layout_passes=False),
        scratch_types=[
            pltpu.VMEM((_C1, _RPT, _W), jnp.float32),
            pltpu.VMEM((_FH, _RC, _W), jnp.float32),
            pltpu.VMEM((_PC,), jnp.int32),
            pltpu.VMEM((_PPT,), jnp.float32),
            pltpu.VMEM((_PPT,), jnp.int32),
            pltpu.VMEM((_NCLS * _FH * 16,), jnp.float32),
            pltpu.VMEM((_NCLS * _FH * 16,), jnp.float32),
            pltpu.VMEM((_NCLS * 16,), jnp.float32),
            pltpu.VMEM((_FPT, _NCLS), jnp.float32),
            pltpu.VMEM((_FPT, _NCLS), jnp.float32),
            pltpu.VMEM((_NCLS,), jnp.float32),
        ],
    )
    _sem, sums, sqs, cnts = sc(lg, emb)

    it = jnp.asarray(is_train, jnp.int32).reshape(1)
    out = pl.pallas_call(
        _epi_body,
        in_specs=[
            pl.BlockSpec(memory_space=pltpu.SMEM),
            pl.BlockSpec((B, C, L), lambda: (0, 0, 0)),
            pl.BlockSpec((B, C, L), lambda: (0, 0, 0)),
            pl.BlockSpec((B * 16, L), lambda: (0, 0)),
            pl.BlockSpec((L, C), lambda: (0, 0)),
            pl.BlockSpec((L, C), lambda: (0, 0)),
            pl.BlockSpec((L, C), lambda: (0, 0)),
            pl.BlockSpec((1, L), lambda: (0, 0)),
        ],
        out_specs=[
            pl.BlockSpec((L, C), lambda: (0, 0)),
            pl.BlockSpec((L, C), lambda: (0, 0)),
            pl.BlockSpec((L, C), lambda: (0, 0)),
            pl.BlockSpec((1, L), lambda: (0, 0)),
        ],
        out_shape=[
            jax.ShapeDtypeStruct((L, C), jnp.float32),
            jax.ShapeDtypeStruct((L, C), jnp.float32),
            jax.ShapeDtypeStruct((L, C), jnp.float32),
            jax.ShapeDtypeStruct((1, L), jnp.float32),
        ],
    )(it, sums, sqs, cnts.reshape(B * 16, L), features, ex, ex2,
      count.reshape(1, L))

    new_features, new_ex, new_ex2, new_count = out
    acc_loss = jnp.zeros((), jnp.float32)
    return (acc_loss, new_features, new_ex, new_ex2, new_count.reshape(L))


# speedup vs baseline: 2.3902x; 1.4031x over previous
import jax
import jax.numpy as jnp
from jax import lax
from jax.experimental import pallas as pl
from jax.experimental.pallas import tpu as pltpu
from jax.experimental.pallas import tpu_sc as plsc

_NCLS = 256
_NF = 256
_H = 128
_W = 128
_NPIX = _H * _W
_PPT = 1024
_RPT = _PPT // _W
_C1 = 32
_PC = 1024
_RC = _PC // _W
_FPT = 16
_FH = 8


def _sc_body(lg_hbm, emb_hbm,
             sem_hbm, sum_hbm, sq_hbm, cnt_hbm,
             semc_v, bvt, bit, cnt_bins, comp_s, comp_q, cnt_c,
             dsem0, dsem1):
    c = lax.axis_index("c")
    s = lax.axis_index("s")
    lane = lax.iota(jnp.int32, 16)
    zf = jnp.zeros((16,), jnp.float32)
    onef = jnp.ones((16,), jnp.float32)
    dsems = (dsem0, dsem1)

    def _z0(i, _):
        cnt_bins[pl.ds(i * 16, 16)] = zf
        return 0
    lax.fori_loop(0, _NCLS * 16 // 16, _z0, 0)

    h0 = s * _RPT

    def phase1(lg_a, lg_b):
        bufs = (lg_a, lg_b)
        ncc = _NCLS // _C1

        def issue(cc):
            return pltpu.async_copy(
                lg_hbm.at[pl.ds(c * _NCLS + cc * _C1, _C1),
                          pl.ds(h0, _RPT), :],
                bufs[cc % 2], dsems[cc % 2])

        pend = issue(0)
        for cc in range(ncc):
            pend.wait()
            if cc + 1 < ncc:
                pend = issue(cc + 1)
            lg_v = bufs[cc % 2]

            def grp_body(g4, _):
                r = g4 // 2
                colb = (g4 % 2) * 64
                if cc == 0:
                    bv = [jnp.full((16,), -jnp.inf, jnp.float32)
                          for _ in range(4)]
                    bi = [jnp.zeros((16,), jnp.int32) for _ in range(4)]
                else:
                    bv = [bvt[pl.ds(g4 * 64 + u * 16, 16)] for u in range(4)]
                    bi = [bit[pl.ds(g4 * 64 + u * 16, 16)] for u in range(4)]
                for cls in range(_C1):
                    civ = jnp.full((16,), cc * _C1 + cls, jnp.int32)
                    for u in range(4):
                        v = lg_v[cls, r, pl.ds(colb + u * 16, 16)]
                        m = v > bv[u]
                        bv[u] = jnp.where(m, v, bv[u])
                        bi[u] = jnp.where(m, civ, bi[u])
                for u in range(4):
                    bvt[pl.ds(g4 * 64 + u * 16, 16)] = bv[u]
                    bit[pl.ds(g4 * 64 + u * 16, 16)] = bi[u]
                return 0

            lax.fori_loop(0, _PPT // 64, grp_body, 0)

    pl.run_scoped(phase1,
                  pltpu.VMEM((_C1, _RPT, _W), jnp.float32),
                  pltpu.VMEM((_C1, _RPT, _W), jnp.float32))

    def hist_body(g, _):
        bi = bit[pl.ds(g * 16, 16)]
        plsc.addupdate_scatter(cnt_bins, [bi * 16 + lane], onef)
        return 0
    lax.fori_loop(0, _PPT // 16, hist_body, 0)
    pltpu.sync_copy(bit, sem_hbm.at[pl.ds(c * _NPIX + s * _PPT, _PPT)])

    def cnt_red(cb, _):
        cidx = (jnp.full((16,), 0, jnp.int32) + cb * 16 + lane) * 16
        acc = zf
        for k in range(16):
            acc = acc + plsc.load_gather(cnt_bins, [cidx + ((k + lane) & 15)])
        cnt_c[pl.ds(cb * 16, 16)] = acc
        return 0
    lax.fori_loop(0, _NCLS // 16, cnt_red, 0)
    pltpu.sync_copy(cnt_c, cnt_hbm.at[pl.ds((c * 16 + s) * _NCLS, _NCLS)])

    plsc.subcore_barrier()

    def phase2(bins_s, bins_q, emb_a, emb_b):
        ebufs = (emb_a, emb_b)
        nchunk = _NPIX // _PC

        for ph in range(2):
            fbase = s * _FPT + ph * _FH

            def _z1(i, _):
                for u in range(4):
                    bins_s[pl.ds((i * 4 + u) * 16, 16)] = zf
                    bins_q[pl.ds((i * 4 + u) * 16, 16)] = zf
                return 0
            lax.fori_loop(0, _NCLS * _FH // 4, _z1, 0)

            def issue(pc):
                return pltpu.async_copy(
                    emb_hbm.at[pl.ds(c * _NF + fbase, _FH),
                               pl.ds(pc * _RC, _RC), :],
                    ebufs[pc % 2], dsems[pc % 2])

            pend = issue(0)
            for pc in range(nchunk):
                pltpu.sync_copy(sem_hbm.at[pl.ds(c * _NPIX + pc * _PC, _PC)],
                                semc_v)
                pend.wait()
                if pc + 1 < nchunk:
                    pend = issue(pc + 1)
                emb_v = ebufs[pc % 2]

                def g_body(g2, _):
                    r0 = (g2 * 2) // 8
                    col0 = ((g2 * 2) % 8) * 16
                    r1 = (g2 * 2 + 1) // 8
                    col1 = ((g2 * 2 + 1) % 8) * 16
                    idx0 = semc_v[pl.ds(g2 * 32, 16)]
                    idx1 = semc_v[pl.ds(g2 * 32 + 16, 16)]
                    base0 = idx0 * (_FH * 16)
                    base1 = idx1 * (_FH * 16)
                    for f in range(_FH):
                        sl = (lane + f) & 15
                        v0 = emb_v[f, r0, pl.ds(col0, 16)]
                        v1 = emb_v[f, r1, pl.ds(col1, 16)]
                        fidx0 = base0 + (f * 16) + sl
                        fidx1 = base1 + (f * 16) + sl
                        plsc.addupdate_scatter(bins_s, [fidx0], v0)
                        plsc.addupdate_scatter(bins_s, [fidx1], v1)
                        plsc.addupdate_scatter(bins_q, [fidx0], v0 * v0)
                        plsc.addupdate_scatter(bins_q, [fidx1], v1 * v1)
                    return 0

                lax.fori_loop(0, _PC // 32, g_body, 0)

            def red_body(cb, _):
                base = cb * (2 * _FH * 16)
                acc_s = zf
                acc_q = zf
                for k in range(16):
                    pk = ((lane // 8) * (_FH * 16) + (lane % 8) * 16
                          + ((k + (lane % 8) + 8 * (lane // 8)) & 15))
                    acc_s = acc_s + plsc.load_gather(bins_s, [base + pk])
                    acc_q = acc_q + plsc.load_gather(bins_q, [base + pk])
                fi = ph * _FH + lane % 8
                ci = cb * 2 + lane // 8
                plsc.store_scatter(comp_s, [fi, ci], acc_s)
                plsc.store_scatter(comp_q, [fi, ci], acc_q)
                return 0

            lax.fori_loop(0, _NCLS // 2, red_body, 0)

    pl.run_scoped(phase2,
                  pltpu.VMEM((_NCLS * _FH * 16,), jnp.float32),
                  pltpu.VMEM((_NCLS * _FH * 16,), jnp.float32),
                  pltpu.VMEM((_FH, _RC, _W), jnp.float32),
                  pltpu.VMEM((_FH, _RC, _W), jnp.float32))

    pltpu.sync_copy(comp_s, sum_hbm.at[c, pl.ds(s * _FPT, _FPT), :])
    pltpu.sync_copy(comp_q, sq_hbm.at[c, pl.ds(s * _FPT, _FPT), :])


def _epi_body(it_ref, s_ref, q_ref, cnt_ref, feat_ref, ex_ref, ex2_ref,
              cntin_ref, nf_ref, nex_ref, nex2_ref, ncnt_ref):
    train = (it_ref[0] != 0).astype(jnp.float32)
    S = jnp.transpose(s_ref[0] + s_ref[1])
    Q = jnp.transpose(q_ref[0] + q_ref[1])
    cnew_row = jnp.sum(cnt_ref[...], axis=0, keepdims=True)
    cold_row = cntin_ref[...]
    ones_row = jnp.ones((1, _NF), jnp.float32)
    dn = (((0,), (0,)), ((), ()))
    cnew = lax.dot_general(cnew_row, ones_row, dn,
                           preferred_element_type=jnp.float32)
    cold = lax.dot_general(cold_row, ones_row, dn,
                           preferred_element_type=jnp.float32)
    feat = feat_ref[...]
    upd_f = (feat * cold + S) / (cold + cnew + 1e-8)
    nf_ref[...] = train * upd_f + (1.0 - train) * feat
    nex_ref[...] = ex_ref[...] + train * S
    nex2_ref[...] = ex2_ref[...] + train * Q
    ncnt_ref[...] = cold_row + train * cnew_row


def kernel(unified_embedding, logits, gt, is_train, dataset_ids, features, ex,
           ex2, count):
    B, C, H, W = unified_embedding.shape
    L = logits.shape[1]
    N = H * W
    emb = unified_embedding.reshape(B * C, H, W)
    lg = logits.reshape(B * L, H, W)

    mesh = plsc.VectorSubcoreMesh(core_axis_name="c", subcore_axis_name="s")
    sc = pl.kernel(
        _sc_body,
        out_type=[
            jax.ShapeDtypeStruct((B * N,), jnp.int32),
            jax.ShapeDtypeStruct((B, C, L), jnp.float32),
            jax.ShapeDtypeStruct((B, C, L), jnp.float32),
            jax.ShapeDtypeStruct((B * 16 * L,), jnp.float32),
        ],
        mesh=mesh,
        compiler_params=pltpu.CompilerParams(needs_layout_passes=False),
        scratch_types=[
            pltpu.VMEM((_PC,), jnp.int32),
            pltpu.VMEM((_PPT,), jnp.float32),
            pltpu.VMEM((_PPT,), jnp.int32),
            pltpu.VMEM((_NCLS * 16,), jnp.float32),
            pltpu.VMEM((_FPT, _NCLS), jnp.float32),
            pltpu.VMEM((_FPT, _NCLS), jnp.float32),
            pltpu.VMEM((_NCLS,), jnp.float32),
            pltpu.SemaphoreType.DMA,
            pltpu.SemaphoreType.DMA,
        ],
    )
    _sem, sums, sqs, cnts = sc(lg, emb)

    it = jnp.asarray(is_train, jnp.int32).reshape(1)
    out = pl.pallas_call(
        _epi_body,
        in_specs=[
            pl.BlockSpec(memory_space=pltpu.SMEM),
            pl.BlockSpec((B, C, L), lambda: (0, 0, 0)),
            pl.BlockSpec((B, C, L), lambda: (0, 0, 0)),
            pl.BlockSpec((B * 16, L), lambda: (0, 0)),
            pl.BlockSpec((L, C), lambda: (0, 0)),
            pl.BlockSpec((L, C), lambda: (0, 0)),
            pl.BlockSpec((L, C), lambda: (0, 0)),
            pl.BlockSpec((1, L), lambda: (0, 0)),
        ],
        out_specs=[
            pl.BlockSpec((L, C), lambda: (0, 0)),
            pl.BlockSpec((L, C), lambda: (0, 0)),
            pl.BlockSpec((L, C), lambda: (0, 0)),
            pl.BlockSpec((1, L), lambda: (0, 0)),
        ],
        out_shape=[
            jax.ShapeDtypeStruct((L, C), jnp.float32),
            jax.ShapeDtypeStruct((L, C), jnp.float32),
            jax.ShapeDtypeStruct((L, C), jnp.float32),
            jax.ShapeDtypeStruct((1, L), jnp.float32),
        ],
    )(it, sums, sqs, cnts.reshape(B * 16, L), features, ex, ex2,
      count.reshape(1, L))

    new_features, new_ex, new_ex2, new_count = out
    acc_loss = jnp.zeros((), jnp.float32)
    return (acc_loss, new_features, new_ex, new_ex2, new_count.reshape(L))


# speedup vs baseline: 2.7079x; 1.1329x over previous
import jax
import jax.numpy as jnp
from jax import lax
from jax.experimental import pallas as pl
from jax.experimental.pallas import tpu as pltpu
from jax.experimental.pallas import tpu_sc as plsc

_NCLS = 256
_NF = 256
_H = 128
_W = 128
_NPIX = _H * _W
_PPT = 1024
_RPT = _PPT // _W
_C1 = 32
_PC = 1024
_RC = _PC // _W
_FPT = 16
_FH = 8


def _sc_body(lg_hbm, emb_hbm,
             sem_hbm, sum_hbm, sq_hbm, cnt_hbm,
             bvt, bit, cnt_bins, comp_s, comp_q, cnt_c,
             dsem0, dsem1):
    c = lax.axis_index("c")
    s = lax.axis_index("s")
    lane = lax.iota(jnp.int32, 16)
    zf = jnp.zeros((16,), jnp.float32)
    onef = jnp.ones((16,), jnp.float32)
    dsems = (dsem0, dsem1)

    def _z0(i, _):
        cnt_bins[pl.ds(i * 16, 16)] = zf
        return 0
    lax.fori_loop(0, _NCLS * 16 // 16, _z0, 0)

    h0 = s * _RPT

    def phase1(lg_a, lg_b):
        bufs = (lg_a, lg_b)
        ncc = _NCLS // _C1

        def issue(cc):
            return pltpu.async_copy(
                lg_hbm.at[pl.ds(c * _NCLS + cc * _C1, _C1),
                          pl.ds(h0, _RPT), :],
                bufs[cc % 2], dsems[cc % 2])

        pend = issue(0)
        for cc in range(ncc):
            pend.wait()
            if cc + 1 < ncc:
                pend = issue(cc + 1)
            lg_v = bufs[cc % 2]

            def grp_body(g4, _):
                r = g4 // 2
                colb = (g4 % 2) * 64
                if cc == 0:
                    bv = [jnp.full((16,), -jnp.inf, jnp.float32)
                          for _ in range(4)]
                    bi = [jnp.zeros((16,), jnp.int32) for _ in range(4)]
                else:
                    bv = [bvt[pl.ds(g4 * 64 + u * 16, 16)] for u in range(4)]
                    bi = [bit[pl.ds(g4 * 64 + u * 16, 16)] for u in range(4)]
                for cls in range(_C1):
                    civ = jnp.full((16,), cc * _C1 + cls, jnp.int32)
                    for u in range(4):
                        v = lg_v[cls, r, pl.ds(colb + u * 16, 16)]
                        m = v > bv[u]
                        bv[u] = jnp.where(m, v, bv[u])
                        bi[u] = jnp.where(m, civ, bi[u])
                for u in range(4):
                    bvt[pl.ds(g4 * 64 + u * 16, 16)] = bv[u]
                    bit[pl.ds(g4 * 64 + u * 16, 16)] = bi[u]
                return 0

            lax.fori_loop(0, _PPT // 64, grp_body, 0)

    pl.run_scoped(phase1,
                  pltpu.VMEM((_C1, _RPT, _W), jnp.float32),
                  pltpu.VMEM((_C1, _RPT, _W), jnp.float32))

    def hist_body(g, _):
        bi = bit[pl.ds(g * 16, 16)]
        plsc.addupdate_scatter(cnt_bins, [bi * 16 + lane], onef)
        return 0
    lax.fori_loop(0, _PPT // 16, hist_body, 0)
    pltpu.sync_copy(bit, sem_hbm.at[pl.ds(c * _NPIX + s * _PPT, _PPT)])

    def cnt_red(cb, _):
        cidx = (jnp.full((16,), 0, jnp.int32) + cb * 16 + lane) * 16
        acc = zf
        for k in range(16):
            acc = acc + plsc.load_gather(cnt_bins, [cidx + ((k + lane) & 15)])
        cnt_c[pl.ds(cb * 16, 16)] = acc
        return 0
    lax.fori_loop(0, _NCLS // 16, cnt_red, 0)
    pltpu.sync_copy(cnt_c, cnt_hbm.at[pl.ds((c * 16 + s) * _NCLS, _NCLS)])

    plsc.subcore_barrier()

    def phase2(bins_s, bins_q, emb_a, emb_b, sem_all):
        ebufs = (emb_a, emb_b)
        nchunk = _NPIX // _PC

        for ph in range(2):
            fbase = s * _FPT + ph * _FH
            pltpu.sync_copy(sem_hbm.at[pl.ds(c * _NPIX, _NPIX)], sem_all)

            def _z1(i, _):
                for u in range(4):
                    bins_s[pl.ds((i * 4 + u) * 16, 16)] = zf
                    bins_q[pl.ds((i * 4 + u) * 16, 16)] = zf
                return 0
            lax.fori_loop(0, _NCLS * _FH // 4, _z1, 0)

            def issue(pc):
                return pltpu.async_copy(
                    emb_hbm.at[pl.ds(c * _NF + fbase, _FH),
                               pl.ds(pc * _RC, _RC), :],
                    ebufs[pc % 2], dsems[pc % 2])

            pend = issue(0)
            for pc in range(nchunk):
                pend.wait()
                if pc + 1 < nchunk:
                    pend = issue(pc + 1)
                emb_v = ebufs[pc % 2]

                def g_body(g2, _):
                    r0 = (g2 * 2) // 8
                    col0 = ((g2 * 2) % 8) * 16
                    r1 = (g2 * 2 + 1) // 8
                    col1 = ((g2 * 2 + 1) % 8) * 16
                    idx0 = sem_all[pl.ds(pc * _PC + g2 * 32, 16)]
                    idx1 = sem_all[pl.ds(pc * _PC + g2 * 32 + 16, 16)]
                    base0 = idx0 * (_FH * 16)
                    base1 = idx1 * (_FH * 16)
                    for f in range(_FH):
                        sl = (lane + f) & 15
                        v0 = emb_v[f, r0, pl.ds(col0, 16)]
                        v1 = emb_v[f, r1, pl.ds(col1, 16)]
                        fidx0 = base0 + (f * 16) + sl
                        fidx1 = base1 + (f * 16) + sl
                        plsc.addupdate_scatter(bins_s, [fidx0], v0)
                        plsc.addupdate_scatter(bins_s, [fidx1], v1)
                        plsc.addupdate_scatter(bins_q, [fidx0], v0 * v0)
                        plsc.addupdate_scatter(bins_q, [fidx1], v1 * v1)
                    return 0

                lax.fori_loop(0, _PC // 32, g_body, 0)

            def red_body(cb, _):
                base = cb * (2 * _FH * 16)
                acc_s = zf
                acc_q = zf
                for k in range(16):
                    pk = ((lane // 8) * (_FH * 16) + (lane % 8) * 16
                          + ((k + (lane % 8) + 8 * (lane // 8)) & 15))
                    acc_s = acc_s + plsc.load_gather(bins_s, [base + pk])
                    acc_q = acc_q + plsc.load_gather(bins_q, [base + pk])
                fi = ph * _FH + lane % 8
                ci = cb * 2 + lane // 8
                plsc.store_scatter(comp_s, [fi, ci], acc_s)
                plsc.store_scatter(comp_q, [fi, ci], acc_q)
                return 0

            lax.fori_loop(0, _NCLS // 2, red_body, 0)

    pl.run_scoped(phase2,
                  pltpu.VMEM((_NCLS * _FH * 16,), jnp.float32),
                  pltpu.VMEM((_NCLS * _FH * 16,), jnp.float32),
                  pltpu.VMEM((_FH, _RC, _W), jnp.float32),
                  pltpu.VMEM((_FH, _RC, _W), jnp.float32),
                  pltpu.VMEM((_NPIX,), jnp.int32))

    pltpu.sync_copy(comp_s, sum_hbm.at[c, pl.ds(s * _FPT, _FPT), :])
    pltpu.sync_copy(comp_q, sq_hbm.at[c, pl.ds(s * _FPT, _FPT), :])


def _epi_body(it_ref, s_ref, q_ref, cnt_ref, feat_ref, ex_ref, ex2_ref,
              cntin_ref, nf_ref, nex_ref, nex2_ref, ncnt_ref):
    train = (it_ref[0] != 0).astype(jnp.float32)
    S = jnp.transpose(s_ref[0] + s_ref[1])
    Q = jnp.transpose(q_ref[0] + q_ref[1])
    cnew_row = jnp.sum(cnt_ref[...], axis=0, keepdims=True)
    cold_row = cntin_ref[...]
    ones_row = jnp.ones((1, _NF), jnp.float32)
    dn = (((0,), (0,)), ((), ()))
    cnew = lax.dot_general(cnew_row, ones_row, dn,
                           preferred_element_type=jnp.float32)
    cold = lax.dot_general(cold_row, ones_row, dn,
                           preferred_element_type=jnp.float32)
    feat = feat_ref[...]
    upd_f = (feat * cold + S) / (cold + cnew + 1e-8)
    nf_ref[...] = train * upd_f + (1.0 - train) * feat
    nex_ref[...] = ex_ref[...] + train * S
    nex2_ref[...] = ex2_ref[...] + train * Q
    ncnt_ref[...] = cold_row + train * cnew_row


def kernel(unified_embedding, logits, gt, is_train, dataset_ids, features, ex,
           ex2, count):
    B, C, H, W = unified_embedding.shape
    L = logits.shape[1]
    N = H * W
    emb = unified_embedding.reshape(B * C, H, W)
    lg = logits.reshape(B * L, H, W)

    mesh = plsc.VectorSubcoreMesh(core_axis_name="c", subcore_axis_name="s")
    sc = pl.kernel(
        _sc_body,
        out_type=[
            jax.ShapeDtypeStruct((B * N,), jnp.int32),
            jax.ShapeDtypeStruct((B, C, L), jnp.float32),
            jax.ShapeDtypeStruct((B, C, L), jnp.float32),
            jax.ShapeDtypeStruct((B * 16 * L,), jnp.float32),
        ],
        mesh=mesh,
        compiler_params=pltpu.CompilerParams(needs_layout_passes=False),
        scratch_types=[
            pltpu.VMEM((_PPT,), jnp.float32),
            pltpu.VMEM((_PPT,), jnp.int32),
            pltpu.VMEM((_NCLS * 16,), jnp.float32),
            pltpu.VMEM((_FPT, _NCLS), jnp.float32),
            pltpu.VMEM((_FPT, _NCLS), jnp.float32),
            pltpu.VMEM((_NCLS,), jnp.float32),
            pltpu.SemaphoreType.DMA,
            pltpu.SemaphoreType.DMA,
        ],
    )
    _sem, sums, sqs, cnts = sc(lg, emb)

    it = jnp.asarray(is_train, jnp.int32).reshape(1)
    out = pl.pallas_call(
        _epi_body,
        in_specs=[
            pl.BlockSpec(memory_space=pltpu.SMEM),
            pl.BlockSpec((B, C, L), lambda: (0, 0, 0)),
            pl.BlockSpec((B, C, L), lambda: (0, 0, 0)),
            pl.BlockSpec((B * 16, L), lambda: (0, 0)),
            pl.BlockSpec((L, C), lambda: (0, 0)),
            pl.BlockSpec((L, C), lambda: (0, 0)),
            pl.BlockSpec((L, C), lambda: (0, 0)),
            pl.BlockSpec((1, L), lambda: (0, 0)),
        ],
        out_specs=[
            pl.BlockSpec((L, C), lambda: (0, 0)),
            pl.BlockSpec((L, C), lambda: (0, 0)),
            pl.BlockSpec((L, C), lambda: (0, 0)),
            pl.BlockSpec((1, L), lambda: (0, 0)),
        ],
        out_shape=[
            jax.ShapeDtypeStruct((L, C), jnp.float32),
            jax.ShapeDtypeStruct((L, C), jnp.float32),
            jax.ShapeDtypeStruct((L, C), jnp.float32),
            jax.ShapeDtypeStruct((1, L), jnp.float32),
        ],
    )(it, sums, sqs, cnts.reshape(B * 16, L), features, ex, ex2,
      count.reshape(1, L))

    new_features, new_ex, new_ex2, new_count = out
    acc_loss = jnp.zeros((), jnp.float32)
    return (acc_loss, new_features, new_ex, new_ex2, new_count.reshape(L))


# speedup vs baseline: 2.7819x; 1.0273x over previous
import jax
import jax.numpy as jnp
from jax import lax
from jax.experimental import pallas as pl
from jax.experimental.pallas import tpu as pltpu
from jax.experimental.pallas import tpu_sc as plsc

_NCLS = 256
_NF = 256
_H = 128
_W = 128
_NPIX = _H * _W
_PPT = 1024
_RPT = _PPT // _W
_C1 = 32
_PC = 2048
_RC = _PC // _W
_FPT = 16
_FH = 8


def _sc_body(lg_hbm, emb_hbm,
             sem_hbm, sum_hbm, sq_hbm, cnt_hbm,
             bvt, bit, cnt_bins, comp_s, comp_q, cnt_c,
             dsem0, dsem1):
    c = lax.axis_index("c")
    s = lax.axis_index("s")
    lane = lax.iota(jnp.int32, 16)
    zf = jnp.zeros((16,), jnp.float32)
    onef = jnp.ones((16,), jnp.float32)
    dsems = (dsem0, dsem1)

    def _z0(i, _):
        cnt_bins[pl.ds(i * 16, 16)] = zf
        return 0
    lax.fori_loop(0, _NCLS * 16 // 16, _z0, 0)

    h0 = s * _RPT

    def phase1(lg_a, lg_b):
        bufs = (lg_a, lg_b)
        ncc = _NCLS // _C1

        def issue(cc):
            return pltpu.async_copy(
                lg_hbm.at[pl.ds(c * _NCLS + cc * _C1, _C1),
                          pl.ds(h0, _RPT), :],
                bufs[cc % 2], dsems[cc % 2])

        pend = issue(0)
        for cc in range(ncc):
            pend.wait()
            if cc + 1 < ncc:
                pend = issue(cc + 1)
            lg_v = bufs[cc % 2]

            def grp_body(g4, _):
                r = g4 // 2
                colb = (g4 % 2) * 64
                if cc == 0:
                    bv = [jnp.full((16,), -jnp.inf, jnp.float32)
                          for _ in range(4)]
                    bi = [jnp.zeros((16,), jnp.int32) for _ in range(4)]
                else:
                    bv = [bvt[pl.ds(g4 * 64 + u * 16, 16)] for u in range(4)]
                    bi = [bit[pl.ds(g4 * 64 + u * 16, 16)] for u in range(4)]
                for cls in range(_C1):
                    civ = jnp.full((16,), cc * _C1 + cls, jnp.int32)
                    for u in range(4):
                        v = lg_v[cls, r, pl.ds(colb + u * 16, 16)]
                        m = v > bv[u]
                        bv[u] = jnp.where(m, v, bv[u])
                        bi[u] = jnp.where(m, civ, bi[u])
                for u in range(4):
                    bvt[pl.ds(g4 * 64 + u * 16, 16)] = bv[u]
                    bit[pl.ds(g4 * 64 + u * 16, 16)] = bi[u]
                return 0

            lax.fori_loop(0, _PPT // 64, grp_body, 0)

    pl.run_scoped(phase1,
                  pltpu.VMEM((_C1, _RPT, _W), jnp.float32),
                  pltpu.VMEM((_C1, _RPT, _W), jnp.float32))

    def hist_body(g, _):
        bi = bit[pl.ds(g * 16, 16)]
        plsc.addupdate_scatter(cnt_bins, [bi * 16 + lane], onef)
        return 0
    lax.fori_loop(0, _PPT // 16, hist_body, 0)
    pltpu.sync_copy(bit, sem_hbm.at[pl.ds(c * _NPIX + s * _PPT, _PPT)])

    def cnt_red(cb, _):
        cidx = (jnp.full((16,), 0, jnp.int32) + cb * 16 + lane) * 16
        acc = zf
        for k in range(16):
            acc = acc + plsc.load_gather(cnt_bins, [cidx + ((k + lane) & 15)])
        cnt_c[pl.ds(cb * 16, 16)] = acc
        return 0
    lax.fori_loop(0, _NCLS // 16, cnt_red, 0)
    pltpu.sync_copy(cnt_c, cnt_hbm.at[pl.ds((c * 16 + s) * _NCLS, _NCLS)])

    plsc.subcore_barrier()

    def phase2(bins_s, bins_q, emb_a, emb_b, sem_all):
        ebufs = (emb_a, emb_b)
        nchunk = _NPIX // _PC

        for ph in range(2):
            fbase = s * _FPT + ph * _FH
            pltpu.sync_copy(sem_hbm.at[pl.ds(c * _NPIX, _NPIX)], sem_all)

            def _z1(i, _):
                for u in range(4):
                    bins_s[pl.ds((i * 4 + u) * 16, 16)] = zf
                    bins_q[pl.ds((i * 4 + u) * 16, 16)] = zf
                return 0
            lax.fori_loop(0, _NCLS * _FH // 4, _z1, 0)

            def issue(pc):
                return pltpu.async_copy(
                    emb_hbm.at[pl.ds(c * _NF + fbase, _FH),
                               pl.ds(pc * _RC, _RC), :],
                    ebufs[pc % 2], dsems[pc % 2])

            pend = issue(0)
            for pc in range(nchunk):
                pend.wait()
                if pc + 1 < nchunk:
                    pend = issue(pc + 1)
                emb_v = ebufs[pc % 2]

                def g_body(g2, _):
                    r0 = (g2 * 2) // 8
                    col0 = ((g2 * 2) % 8) * 16
                    r1 = (g2 * 2 + 1) // 8
                    col1 = ((g2 * 2 + 1) % 8) * 16
                    idx0 = sem_all[pl.ds(pc * _PC + g2 * 32, 16)]
                    idx1 = sem_all[pl.ds(pc * _PC + g2 * 32 + 16, 16)]
                    base0 = idx0 * (_FH * 16)
                    base1 = idx1 * (_FH * 16)
                    for f in range(_FH):
                        sl = (lane + f) & 15
                        v0 = emb_v[f, r0, pl.ds(col0, 16)]
                        v1 = emb_v[f, r1, pl.ds(col1, 16)]
                        fidx0 = base0 + (f * 16) + sl
                        fidx1 = base1 + (f * 16) + sl
                        plsc.addupdate_scatter(bins_s, [fidx0], v0)
                        plsc.addupdate_scatter(bins_s, [fidx1], v1)
                        plsc.addupdate_scatter(bins_q, [fidx0], v0 * v0)
                        plsc.addupdate_scatter(bins_q, [fidx1], v1 * v1)
                    return 0

                lax.fori_loop(0, _PC // 32, g_body, 0)

            def red_body(cb, _):
                base = cb * (2 * _FH * 16)
                acc_s = zf
                acc_q = zf
                for k in range(16):
                    pk = ((lane // 8) * (_FH * 16) + (lane % 8) * 16
                          + ((k + (lane % 8) + 8 * (lane // 8)) & 15))
                    acc_s = acc_s + plsc.load_gather(bins_s, [base + pk])
                    acc_q = acc_q + plsc.load_gather(bins_q, [base + pk])
                fi = ph * _FH + lane % 8
                ci = cb * 2 + lane // 8
                plsc.store_scatter(comp_s, [fi, ci], acc_s)
                plsc.store_scatter(comp_q, [fi, ci], acc_q)
                return 0

            lax.fori_loop(0, _NCLS // 2, red_body, 0)

    pl.run_scoped(phase2,
                  pltpu.VMEM((_NCLS * _FH * 16,), jnp.float32),
                  pltpu.VMEM((_NCLS * _FH * 16,), jnp.float32),
                  pltpu.VMEM((_FH, _RC, _W), jnp.float32),
                  pltpu.VMEM((_FH, _RC, _W), jnp.float32),
                  pltpu.VMEM((_NPIX,), jnp.int32))

    pltpu.sync_copy(comp_s, sum_hbm.at[c, pl.ds(s * _FPT, _FPT), :])
    pltpu.sync_copy(comp_q, sq_hbm.at[c, pl.ds(s * _FPT, _FPT), :])


def _epi_body(it_ref, s_ref, q_ref, cnt_ref, feat_ref, ex_ref, ex2_ref,
              cntin_ref, nf_ref, nex_ref, nex2_ref, ncnt_ref):
    train = (it_ref[0] != 0).astype(jnp.float32)
    S = jnp.transpose(s_ref[0] + s_ref[1])
    Q = jnp.transpose(q_ref[0] + q_ref[1])
    cnew_row = jnp.sum(cnt_ref[...], axis=0, keepdims=True)
    cold_row = cntin_ref[...]
    ones_row = jnp.ones((1, _NF), jnp.float32)
    dn = (((0,), (0,)), ((), ()))
    cnew = lax.dot_general(cnew_row, ones_row, dn,
                           preferred_element_type=jnp.float32)
    cold = lax.dot_general(cold_row, ones_row, dn,
                           preferred_element_type=jnp.float32)
    feat = feat_ref[...]
    upd_f = (feat * cold + S) / (cold + cnew + 1e-8)
    nf_ref[...] = train * upd_f + (1.0 - train) * feat
    nex_ref[...] = ex_ref[...] + train * S
    nex2_ref[...] = ex2_ref[...] + train * Q
    ncnt_ref[...] = cold_row + train * cnew_row


def kernel(unified_embedding, logits, gt, is_train, dataset_ids, features, ex,
           ex2, count):
    B, C, H, W = unified_embedding.shape
    L = logits.shape[1]
    N = H * W
    emb = unified_embedding.reshape(B * C, H, W)
    lg = logits.reshape(B * L, H, W)

    mesh = plsc.VectorSubcoreMesh(core_axis_name="c", subcore_axis_name="s")
    sc = pl.kernel(
        _sc_body,
        out_type=[
            jax.ShapeDtypeStruct((B * N,), jnp.int32),
            jax.ShapeDtypeStruct((B, C, L), jnp.float32),
            jax.ShapeDtypeStruct((B, C, L), jnp.float32),
            jax.ShapeDtypeStruct((B * 16 * L,), jnp.float32),
        ],
        mesh=mesh,
        compiler_params=pltpu.CompilerParams(needs_layout_passes=False),
        scratch_types=[
            pltpu.VMEM((_PPT,), jnp.float32),
            pltpu.VMEM((_PPT,), jnp.int32),
            pltpu.VMEM((_NCLS * 16,), jnp.float32),
            pltpu.VMEM((_FPT, _NCLS), jnp.float32),
            pltpu.VMEM((_FPT, _NCLS), jnp.float32),
            pltpu.VMEM((_NCLS,), jnp.float32),
            pltpu.SemaphoreType.DMA,
            pltpu.SemaphoreType.DMA,
        ],
    )
    _sem, sums, sqs, cnts = sc(lg, emb)

    it = jnp.asarray(is_train, jnp.int32).reshape(1)
    out = pl.pallas_call(
        _epi_body,
        in_specs=[
            pl.BlockSpec(memory_space=pltpu.SMEM),
            pl.BlockSpec((B, C, L), lambda: (0, 0, 0)),
            pl.BlockSpec((B, C, L), lambda: (0, 0, 0)),
            pl.BlockSpec((B * 16, L), lambda: (0, 0)),
            pl.BlockSpec((L, C), lambda: (0, 0)),
            pl.BlockSpec((L, C), lambda: (0, 0)),
            pl.BlockSpec((L, C), lambda: (0, 0)),
            pl.BlockSpec((1, L), lambda: (0, 0)),
        ],
        out_specs=[
            pl.BlockSpec((L, C), lambda: (0, 0)),
            pl.BlockSpec((L, C), lambda: (0, 0)),
            pl.BlockSpec((L, C), lambda: (0, 0)),
            pl.BlockSpec((1, L), lambda: (0, 0)),
        ],
        out_shape=[
            jax.ShapeDtypeStruct((L, C), jnp.float32),
            jax.ShapeDtypeStruct((L, C), jnp.float32),
            jax.ShapeDtypeStruct((L, C), jnp.float32),
            jax.ShapeDtypeStruct((1, L), jnp.float32),
        ],
    )(it, sums, sqs, cnts.reshape(B * 16, L), features, ex, ex2,
      count.reshape(1, L))

    new_features, new_ex, new_ex2, new_count = out
    acc_loss = jnp.zeros((), jnp.float32)
    return (acc_loss, new_features, new_ex, new_ex2, new_count.reshape(L))


# speedup vs baseline: 2.9318x; 1.0539x over previous
import jax
import jax.numpy as jnp
from jax import lax
from jax.experimental import pallas as pl
from jax.experimental.pallas import tpu as pltpu
from jax.experimental.pallas import tpu_sc as plsc

_NCLS = 256
_NF = 256
_H = 128
_W = 128
_NPIX = _H * _W
_PPT = 1024
_RPT = _PPT // _W
_C1 = 32
_PC = 2048
_RC = _PC // _W
_FPT = 16
_FH = 8


def _sc_body(lg_hbm, emb_hbm,
             sem_hbm, sum_hbm, sq_hbm, cnt_hbm,
             bvt, bit, cnt_bins, comp_s, comp_q, cnt_c,
             dsem0, dsem1):
    c = lax.axis_index("c")
    s = lax.axis_index("s")
    lane = lax.iota(jnp.int32, 16)
    zf = jnp.zeros((16,), jnp.float32)
    onef = jnp.ones((16,), jnp.float32)
    dsems = (dsem0, dsem1)

    h0 = s * _RPT

    def phase1(lg_a, lg_b):
        bufs = (lg_a, lg_b)
        ncc = _NCLS // _C1

        def issue(cc):
            return pltpu.async_copy(
                lg_hbm.at[pl.ds(c * _NCLS + cc * _C1, _C1),
                          pl.ds(h0, _RPT), :],
                bufs[cc % 2], dsems[cc % 2])

        pend = issue(0)

        def _z0(i, _):
            cnt_bins[pl.ds(i * 16, 16)] = zf
            return 0
        lax.fori_loop(0, _NCLS * 16 // 16, _z0, 0)

        for cc in range(ncc):
            pend.wait()
            if cc + 1 < ncc:
                pend = issue(cc + 1)
            lg_v = bufs[cc % 2]

            def grp_body(g4, _):
                r = g4 // 2
                colb = (g4 % 2) * 64
                if cc == 0:
                    bv = [jnp.full((16,), -jnp.inf, jnp.float32)
                          for _ in range(4)]
                    bi = [jnp.zeros((16,), jnp.int32) for _ in range(4)]
                else:
                    bv = [bvt[pl.ds(g4 * 64 + u * 16, 16)] for u in range(4)]
                    bi = [bit[pl.ds(g4 * 64 + u * 16, 16)] for u in range(4)]
                for cls in range(_C1):
                    civ = jnp.full((16,), cc * _C1 + cls, jnp.int32)
                    for u in range(4):
                        v = lg_v[cls, r, pl.ds(colb + u * 16, 16)]
                        m = v > bv[u]
                        bv[u] = jnp.where(m, v, bv[u])
                        bi[u] = jnp.where(m, civ, bi[u])
                for u in range(4):
                    bvt[pl.ds(g4 * 64 + u * 16, 16)] = bv[u]
                    bit[pl.ds(g4 * 64 + u * 16, 16)] = bi[u]
                return 0

            lax.fori_loop(0, _PPT // 64, grp_body, 0)

    pl.run_scoped(phase1,
                  pltpu.VMEM((_C1, _RPT, _W), jnp.float32),
                  pltpu.VMEM((_C1, _RPT, _W), jnp.float32))

    def hist_body(g, _):
        bi = bit[pl.ds(g * 16, 16)]
        plsc.addupdate_scatter(cnt_bins, [bi * 16 + lane], onef)
        return 0
    lax.fori_loop(0, _PPT // 16, hist_body, 0)
    pltpu.sync_copy(bit, sem_hbm.at[pl.ds(c * _NPIX + s * _PPT, _PPT)])

    def cnt_red(cb, _):
        cidx = (jnp.full((16,), 0, jnp.int32) + cb * 16 + lane) * 16
        acc = zf
        for k in range(16):
            acc = acc + plsc.load_gather(cnt_bins, [cidx + ((k + lane) & 15)])
        cnt_c[pl.ds(cb * 16, 16)] = acc
        return 0
    lax.fori_loop(0, _NCLS // 16, cnt_red, 0)
    pltpu.sync_copy(cnt_c, cnt_hbm.at[pl.ds((c * 16 + s) * _NCLS, _NCLS)])

    plsc.subcore_barrier()

    def phase2(bins_s, bins_q, emb_a, emb_b, sem_all):
        ebufs = (emb_a, emb_b)
        nchunk = _NPIX // _PC

        for ph in range(2):
            fbase = s * _FPT + ph * _FH

            def issue(pc):
                return pltpu.async_copy(
                    emb_hbm.at[pl.ds(c * _NF + fbase, _FH),
                               pl.ds(pc * _RC, _RC), :],
                    ebufs[pc % 2], dsems[pc % 2])

            pend = issue(0)
            if ph == 0:
                pltpu.sync_copy(sem_hbm.at[pl.ds(c * _NPIX, _NPIX)], sem_all)

            def _z1(i, _):
                for u in range(4):
                    bins_s[pl.ds((i * 4 + u) * 16, 16)] = zf
                    bins_q[pl.ds((i * 4 + u) * 16, 16)] = zf
                return 0
            lax.fori_loop(0, _NCLS * _FH // 4, _z1, 0)

            for pc in range(nchunk):
                pend.wait()
                if pc + 1 < nchunk:
                    pend = issue(pc + 1)
                emb_v = ebufs[pc % 2]

                def g_body(g2, _):
                    r0 = (g2 * 2) // 8
                    col0 = ((g2 * 2) % 8) * 16
                    r1 = (g2 * 2 + 1) // 8
                    col1 = ((g2 * 2 + 1) % 8) * 16
                    idx0 = sem_all[pl.ds(pc * _PC + g2 * 32, 16)]
                    idx1 = sem_all[pl.ds(pc * _PC + g2 * 32 + 16, 16)]
                    base0 = idx0 * (_FH * 16)
                    base1 = idx1 * (_FH * 16)
                    for f in range(_FH):
                        sl = (lane + f) & 15
                        v0 = emb_v[f, r0, pl.ds(col0, 16)]
                        v1 = emb_v[f, r1, pl.ds(col1, 16)]
                        fidx0 = base0 + (f * 16) + sl
                        fidx1 = base1 + (f * 16) + sl
                        plsc.addupdate_scatter(bins_s, [fidx0], v0)
                        plsc.addupdate_scatter(bins_s, [fidx1], v1)
                        plsc.addupdate_scatter(bins_q, [fidx0], v0 * v0)
                        plsc.addupdate_scatter(bins_q, [fidx1], v1 * v1)
                    return 0

                lax.fori_loop(0, _PC // 32, g_body, 0)

            def red_body(cb, _):
                base = cb * (2 * _FH * 16)
                acc_s = zf
                acc_q = zf
                for k in range(16):
                    pk = ((lane // 8) * (_FH * 16) + (lane % 8) * 16
                          + ((k + (lane % 8) + 8 * (lane // 8)) & 15))
                    acc_s = acc_s + plsc.load_gather(bins_s, [base + pk])
                    acc_q = acc_q + plsc.load_gather(bins_q, [base + pk])
                fi = ph * _FH + lane % 8
                ci = cb * 2 + lane // 8
                plsc.store_scatter(comp_s, [fi, ci], acc_s)
                plsc.store_scatter(comp_q, [fi, ci], acc_q)
                return 0

            lax.fori_loop(0, _NCLS // 2, red_body, 0)

    pl.run_scoped(phase2,
                  pltpu.VMEM((_NCLS * _FH * 16,), jnp.float32),
                  pltpu.VMEM((_NCLS * _FH * 16,), jnp.float32),
                  pltpu.VMEM((_FH, _RC, _W), jnp.float32),
                  pltpu.VMEM((_FH, _RC, _W), jnp.float32),
                  pltpu.VMEM((_NPIX,), jnp.int32))

    pltpu.sync_copy(comp_s, sum_hbm.at[c, pl.ds(s * _FPT, _FPT), :])
    pltpu.sync_copy(comp_q, sq_hbm.at[c, pl.ds(s * _FPT, _FPT), :])


def _epi_body(it_ref, s_ref, q_ref, cnt_ref, feat_ref, ex_ref, ex2_ref,
              cntin_ref, nf_ref, nex_ref, nex2_ref, ncnt_ref):
    train = (it_ref[0] != 0).astype(jnp.float32)
    S = jnp.transpose(s_ref[0] + s_ref[1])
    Q = jnp.transpose(q_ref[0] + q_ref[1])
    cnew_row = jnp.sum(cnt_ref[...], axis=0, keepdims=True)
    cold_row = cntin_ref[...]
    ones_row = jnp.ones((1, _NF), jnp.float32)
    dn = (((0,), (0,)), ((), ()))
    cnew = lax.dot_general(cnew_row, ones_row, dn,
                           preferred_element_type=jnp.float32)
    cold = lax.dot_general(cold_row, ones_row, dn,
                           preferred_element_type=jnp.float32)
    feat = feat_ref[...]
    upd_f = (feat * cold + S) / (cold + cnew + 1e-8)
    nf_ref[...] = train * upd_f + (1.0 - train) * feat
    nex_ref[...] = ex_ref[...] + train * S
    nex2_ref[...] = ex2_ref[...] + train * Q
    ncnt_ref[...] = cold_row + train * cnew_row


def kernel(unified_embedding, logits, gt, is_train, dataset_ids, features, ex,
           ex2, count):
    B, C, H, W = unified_embedding.shape
    L = logits.shape[1]
    N = H * W
    emb = unified_embedding.reshape(B * C, H, W)
    lg = logits.reshape(B * L, H, W)

    mesh = plsc.VectorSubcoreMesh(core_axis_name="c", subcore_axis_name="s")
    sc = pl.kernel(
        _sc_body,
        out_type=[
            jax.ShapeDtypeStruct((B * N,), jnp.int32),
            jax.ShapeDtypeStruct((B, C, L), jnp.float32),
            jax.ShapeDtypeStruct((B, C, L), jnp.float32),
            jax.ShapeDtypeStruct((B * 16 * L,), jnp.float32),
        ],
        mesh=mesh,
        compiler_params=pltpu.CompilerParams(needs_layout_passes=False),
        scratch_types=[
            pltpu.VMEM((_PPT,), jnp.float32),
            pltpu.VMEM((_PPT,), jnp.int32),
            pltpu.VMEM((_NCLS * 16,), jnp.float32),
            pltpu.VMEM((_FPT, _NCLS), jnp.float32),
            pltpu.VMEM((_FPT, _NCLS), jnp.float32),
            pltpu.VMEM((_NCLS,), jnp.float32),
            pltpu.SemaphoreType.DMA,
            pltpu.SemaphoreType.DMA,
        ],
    )
    _sem, sums, sqs, cnts = sc(lg, emb)

    it = jnp.asarray(is_train, jnp.int32).reshape(1)
    out = pl.pallas_call(
        _epi_body,
        in_specs=[
            pl.BlockSpec(memory_space=pltpu.SMEM),
            pl.BlockSpec((B, C, L), lambda: (0, 0, 0)),
            pl.BlockSpec((B, C, L), lambda: (0, 0, 0)),
            pl.BlockSpec((B * 16, L), lambda: (0, 0)),
            pl.BlockSpec((L, C), lambda: (0, 0)),
            pl.BlockSpec((L, C), lambda: (0, 0)),
            pl.BlockSpec((L, C), lambda: (0, 0)),
            pl.BlockSpec((1, L), lambda: (0, 0)),
        ],
        out_specs=[
            pl.BlockSpec((L, C), lambda: (0, 0)),
            pl.BlockSpec((L, C), lambda: (0, 0)),
            pl.BlockSpec((L, C), lambda: (0, 0)),
            pl.BlockSpec((1, L), lambda: (0, 0)),
        ],
        out_shape=[
            jax.ShapeDtypeStruct((L, C), jnp.float32),
            jax.ShapeDtypeStruct((L, C), jnp.float32),
            jax.ShapeDtypeStruct((L, C), jnp.float32),
            jax.ShapeDtypeStruct((1, L), jnp.float32),
        ],
    )(it, sums, sqs, cnts.reshape(B * 16, L), features, ex, ex2,
      count.reshape(1, L))

    new_features, new_ex, new_ex2, new_count = out
    acc_loss = jnp.zeros((), jnp.float32)
    return (acc_loss, new_features, new_ex, new_ex2, new_count.reshape(L))
